# baseline (speedup 1.0000x reference)
import jax
import jax.numpy as jnp
from jax.experimental import pallas as pl
from jax.experimental.pallas import tpu as pltpu

N = 10000
BM = 512
GRID = -(-N // BM)
NPAD = GRID * BM

_CONTRACT_LAST = (((1,), (1,)), ((), ()))
_CONTRACT_00 = (((0,), (0,)), ((), ()))


def _gcn(adjA_ref, adjB_ref, x_ref, W1_ref, b1c_ref, W2_ref, b2r_ref,
         out_ref, sT_ref, gT_ref):
    p = pl.program_id(0)
    i = pl.program_id(1)
    HB = BM // 2

    @pl.when((p == 0) & (i == 0))
    def _():
        s = jnp.dot(x_ref[:], W1_ref[:], preferred_element_type=jnp.float32)
        sT_ref[:] = s.T

    @pl.when(p == 0)
    def _():
        for k, a_ref in enumerate((adjA_ref, adjB_ref)):
            hT = jax.lax.dot_general(sT_ref[:], a_ref[:], _CONTRACT_LAST,
                                     preferred_element_type=jnp.float32)
            hT = jnp.maximum(hT + b1c_ref[:], 0.0)
            gT_ref[:, pl.ds(i * BM + k * HB, HB)] = jax.lax.dot_general(
                W2_ref[:], hT, _CONTRACT_00,
                preferred_element_type=jnp.float32)

    @pl.when(p == 1)
    def _():
        for k, a_ref in enumerate((adjA_ref, adjB_ref)):
            oT = jax.lax.dot_general(gT_ref[:, :N], a_ref[:],
                                     _CONTRACT_LAST,
                                     preferred_element_type=jnp.float32)
            out_ref[pl.ds(k * HB, HB), :] = oT.T + b2r_ref[:]


def kernel(x, adj, W1, b1, W2, b2):
    nfeat = x.shape[1]
    nhid = W1.shape[1]
    nclass = W2.shape[1]
    b1c = b1.reshape(nhid, 1)
    b2r = b2.reshape(1, nclass)

    full = lambda shape: pl.BlockSpec(shape, lambda p, i: (0, 0))

    out = pl.pallas_call(
        _gcn,
        grid=(2, GRID),
        in_specs=[
            pl.BlockSpec((BM // 2, N), lambda p, i: (2 * i, 0)),
            pl.BlockSpec((BM // 2, N), lambda p, i: (2 * i + 1, 0)),
            full((N, nfeat)),
            full((nfeat, nhid)),
            full((nhid, 1)),
            full((nhid, nclass)),
            full((1, nclass)),
        ],
        out_specs=pl.BlockSpec((BM, nclass), lambda p, i: (p * i, 0)),
        out_shape=jax.ShapeDtypeStruct((N, nclass), jnp.float32),
        scratch_shapes=[
            pltpu.VMEM((nhid, N), jnp.float32),
            pltpu.VMEM((nclass, NPAD), jnp.float32),
        ],
        compiler_params=pltpu.CompilerParams(
            dimension_semantics=("arbitrary", "arbitrary")),
    )(adj, adj, x, W1, b1c, W2, b2r)

    return out


# speedup vs baseline: 1.0615x; 1.0615x over previous
import jax
import jax.numpy as jnp
from jax.experimental import pallas as pl
from jax.experimental.pallas import tpu as pltpu

N = 10000
BM = 256
NBLK = -(-N // BM)
NPAD = NBLK * BM
CACHE_SLABS = 6
CACHE_ROWS = CACHE_SLABS * BM
NSTREAM = NBLK - CACHE_SLABS

_CONTRACT_LAST = (((1,), (1,)), ((), ()))
_CONTRACT_00 = (((0,), (0,)), ((), ()))


def _gcn(adj_ref, x_ref, W1_ref, b1c_ref, W2_ref, b2r_ref, out_ref,
         sT_ref, gT_ref, cache_ref):
    p = pl.program_id(0)
    i = pl.program_id(1)

    @pl.when((p == 0) & (i == 0))
    def _():
        s = jnp.dot(x_ref[:], W1_ref[:], preferred_element_type=jnp.float32)
        sT_ref[:] = s.T

    @pl.when(p == 0)
    def _():
        hT = jax.lax.dot_general(sT_ref[:], adj_ref[:], _CONTRACT_LAST,
                                 preferred_element_type=jnp.float32)
        hT = jnp.maximum(hT + b1c_ref[:], 0.0)
        gT_ref[:, pl.ds(i * BM, BM)] = jax.lax.dot_general(
            W2_ref[:], hT, _CONTRACT_00, preferred_element_type=jnp.float32)

        @pl.when(i < CACHE_SLABS)
        def _():
            cache_ref[pl.ds(i * BM, BM), :] = adj_ref[:].astype(jnp.bfloat16)

    @pl.when(p == 1)
    def _():
        @pl.when(i < NSTREAM)
        def _():
            oT = jax.lax.dot_general(gT_ref[:, :N], adj_ref[:],
                                     _CONTRACT_LAST,
                                     preferred_element_type=jnp.float32)
            out_ref[:] = oT.T + b2r_ref[:]

        @pl.when(i >= NSTREAM)
        def _():
            j = i - NSTREAM
            gT16 = gT_ref[:, :N].astype(jnp.bfloat16)
            oT = jax.lax.dot_general(
                gT16, cache_ref[pl.ds(j * BM, BM), :], _CONTRACT_LAST,
                preferred_element_type=jnp.float32)
            out_ref[:] = oT.T + b2r_ref[:]


def kernel(x, adj, W1, b1, W2, b2):
    nfeat = x.shape[1]
    nhid = W1.shape[1]
    nclass = W2.shape[1]
    b1c = b1.reshape(nhid, 1)
    b2r = b2.reshape(1, nclass)

    full = lambda shape: pl.BlockSpec(shape, lambda p, i: (0, 0))

    def adj_idx(p, i):
        return (jnp.where(p == 0, i,
                          jnp.minimum(i + CACHE_SLABS, NBLK - 1)), 0)

    def out_idx(p, i):
        return (jnp.where(p == 0, CACHE_SLABS,
                          jnp.where(i < NSTREAM, i + CACHE_SLABS,
                                    i - NSTREAM)), 0)

    out = pl.pallas_call(
        _gcn,
        grid=(2, NBLK),
        in_specs=[
            pl.BlockSpec((BM, N), adj_idx),
            full((N, nfeat)),
            full((nfeat, nhid)),
            full((nhid, 1)),
            full((nhid, nclass)),
            full((1, nclass)),
        ],
        out_specs=pl.BlockSpec((BM, nclass), out_idx),
        out_shape=jax.ShapeDtypeStruct((N, nclass), jnp.float32),
        scratch_shapes=[
            pltpu.VMEM((nhid, N), jnp.float32),
            pltpu.VMEM((nclass, NPAD), jnp.float32),
            pltpu.VMEM((CACHE_ROWS, N), jnp.bfloat16),
        ],
        compiler_params=pltpu.CompilerParams(
            dimension_semantics=("arbitrary", "arbitrary")),
    )(adj, x, W1, b1c, W2, b2r)

    return out


# speedup vs baseline: 1.0780x; 1.0155x over previous
import jax
import jax.numpy as jnp
from jax.experimental import pallas as pl
from jax.experimental.pallas import tpu as pltpu

N = 10000
BM = 256
NBLK = -(-N // BM)
NPAD = NBLK * BM
CACHE_SLABS = 6
CACHE_ROWS = CACHE_SLABS * BM
NSTREAM = NBLK - CACHE_SLABS

_CONTRACT_LAST = (((1,), (1,)), ((), ()))
_CONTRACT_00 = (((0,), (0,)), ((), ()))


def _gcn(adj_ref, x_ref, W1_ref, b1c_ref, W2_ref, b2r_ref, out_ref,
         sT_ref, gT_ref, cache_ref, outcT_ref):
    p = pl.program_id(0)
    i = pl.program_id(1)

    @pl.when((p == 0) & (i == 0))
    def _():
        s = jnp.dot(x_ref[:], W1_ref[:], preferred_element_type=jnp.float32)
        sT_ref[:] = s.T

    @pl.when(p == 0)
    def _():
        hT = jax.lax.dot_general(sT_ref[:], adj_ref[:], _CONTRACT_LAST,
                                 preferred_element_type=jnp.float32)
        hT = jnp.maximum(hT + b1c_ref[:], 0.0)
        gT_ref[:, pl.ds(i * BM, BM)] = jax.lax.dot_general(
            W2_ref[:], hT, _CONTRACT_00, preferred_element_type=jnp.float32)

        @pl.when(i < CACHE_SLABS)
        def _():
            cache_ref[pl.ds(i * BM, BM), :] = adj_ref[:].astype(jnp.bfloat16)

    @pl.when(p == 1)
    def _():
        @pl.when(i < NSTREAM)
        def _():
            oT = jax.lax.dot_general(gT_ref[:, :N], adj_ref[:],
                                     _CONTRACT_LAST,
                                     preferred_element_type=jnp.float32)
            out_ref[:] = oT.T + b2r_ref[:]

        @pl.when(i < CACHE_SLABS)
        def _():
            gT16 = gT_ref[:, :N].astype(jnp.bfloat16)
            outcT_ref[:, pl.ds(i * BM, BM)] = jax.lax.dot_general(
                gT16, cache_ref[pl.ds(i * BM, BM), :], _CONTRACT_LAST,
                preferred_element_type=jnp.float32)

        @pl.when(i >= NSTREAM)
        def _():
            j = i - NSTREAM
            out_ref[:] = outcT_ref[:, pl.ds(j * BM, BM)].T + b2r_ref[:]


def kernel(x, adj, W1, b1, W2, b2):
    nfeat = x.shape[1]
    nhid = W1.shape[1]
    nclass = W2.shape[1]
    b1c = b1.reshape(nhid, 1)
    b2r = b2.reshape(1, nclass)

    full = lambda shape: pl.BlockSpec(shape, lambda p, i: (0, 0))

    def adj_idx(p, i):
        return (jnp.where(p == 0, i,
                          jnp.minimum(i + CACHE_SLABS, NBLK - 1)), 0)

    def out_idx(p, i):
        return (jnp.where(p == 0, CACHE_SLABS,
                          jnp.where(i < NSTREAM, i + CACHE_SLABS,
                                    i - NSTREAM)), 0)

    out = pl.pallas_call(
        _gcn,
        grid=(2, NBLK),
        in_specs=[
            pl.BlockSpec((BM, N), adj_idx),
            full((N, nfeat)),
            full((nfeat, nhid)),
            full((nhid, 1)),
            full((nhid, nclass)),
            full((1, nclass)),
        ],
        out_specs=pl.BlockSpec((BM, nclass), out_idx),
        out_shape=jax.ShapeDtypeStruct((N, nclass), jnp.float32),
        scratch_shapes=[
            pltpu.VMEM((nhid, N), jnp.float32),
            pltpu.VMEM((nclass, NPAD), jnp.float32),
            pltpu.VMEM((CACHE_ROWS, N), jnp.bfloat16),
            pltpu.VMEM((nclass, CACHE_ROWS), jnp.float32),
        ],
        compiler_params=pltpu.CompilerParams(
            dimension_semantics=("arbitrary", "arbitrary")),
    )(adj, x, W1, b1c, W2, b2r)

    return out
